# Initial kernel scaffold; baseline (speedup 1.0000x reference)
import jax
import jax.numpy as jnp
from jax.experimental import pallas as pl


def kernel(x, edge_index, node_depth, z, conv1_W, conv1_b, conv2_W, conv2_b, mlp_W1, mlp_b1, mlp_W2, mlp_b2, Wsrc, Wtgt, inv_W1, inv_b1, inv_W2, inv_b2):
    raise NotImplementedError("write your pallas kernel here")


# speedup vs baseline: 3.7337x; 3.7337x over previous
import functools

import jax
import jax.numpy as jnp
from jax import lax
from jax.experimental import pallas as pl
from jax.experimental.pallas import tpu as pltpu
from jax.experimental.pallas import tpu_sc as plsc

N = 4096
E = 65536
H = 256
Z = 128
NEG = -1e9

NC = 2
NS = 16
NW = NC * NS
CH = 128
EPW = E // NW
NCHUNK = EPW // CH
RPT = N // NS

def _sc_mesh():
    return plsc.VectorSubcoreMesh(
        core_axis_name="c", subcore_axis_name="s",
        num_cores=NC, num_subcores=NS)


def _worker_id():
    return lax.axis_index("s") * NC + lax.axis_index("c")


def _sc_degree(dst):
    zeros = jnp.zeros((N * 16,), jnp.float32)

    @functools.partial(
        pl.kernel,
        out_type=jax.ShapeDtypeStruct((NW, N), jnp.float32),
        mesh=_sc_mesh(),
        compiler_params=pltpu.CompilerParams(needs_layout_passes=False),
        scratch_types=[
            pltpu.VMEM((EPW,), jnp.int32),
            pltpu.VMEM((N * 16,), jnp.float32),
            pltpu.VMEM((N,), jnp.float32),
        ],
    )
    def deg_kernel(dst_hbm, zeros_hbm, out_hbm, idx_v, acc, red_v):
        wid = _worker_id()
        pltpu.sync_copy(zeros_hbm, acc)
        pltpu.sync_copy(dst_hbm.at[pl.ds(wid * EPW, EPW)], idx_v)
        cidx = lax.iota(jnp.int32, 16)
        ones = jnp.ones((16,), jnp.float32)

        def grp(i, carry):
            dvec = idx_v[pl.ds(i * 16, 16)]
            plsc.addupdate_scatter(acc, [dvec * 16 + cidx], ones)
            return carry

        lax.fori_loop(0, EPW // 16, grp, 0)

        def redgrp(v0, carry):
            r = jnp.zeros((16,), jnp.float32)
            for k in range(16):
                w = acc[pl.ds(v0 * 256 + k * 16, 16)]
                r = r + jnp.where(cidx == k, jnp.sum(w), 0.0)
            red_v[pl.ds(v0 * 16, 16)] = r
            return carry

        lax.fori_loop(0, N // 16, redgrp, 0)
        pltpu.sync_copy(red_v, out_hbm.at[wid])

    return deg_kernel(dst, zeros)


EPH = E // 2
SEG = 8192
NSEG = EPH // SEG
CPS = SEG // CH


def _sc_segsum(vals_t, src, dst):
    zeros = jnp.zeros((N * 16,), jnp.float32)

    @functools.partial(
        pl.kernel,
        out_type=jax.ShapeDtypeStruct((2, 16, N * 16), jnp.float32),
        mesh=_sc_mesh(),
        compiler_params=pltpu.CompilerParams(
            needs_layout_passes=False, use_tc_tiling_on_sc=False),
        scratch_types=[
            pltpu.VMEM((SEG,), jnp.int32),
            pltpu.VMEM((SEG,), jnp.int32),
            pltpu.VMEM((CH, 16), jnp.float32),
            pltpu.VMEM((CH, 16), jnp.float32),
            pltpu.VMEM((N * 16,), jnp.float32),
            pltpu.SemaphoreType.DMA,
            pltpu.SemaphoreType.DMA,
        ],
    )
    def seg_kernel(vals_hbm, src_hbm, dst_hbm, zeros_hbm, out_hbm,
                   si_v, di_v, rva, rvb, acc, sema, semb):
        wid = _worker_id()
        g = wid % 16
        hf = wid // 16
        pltpu.sync_copy(zeros_hbm, acc)
        cidx = lax.iota(jnp.int32, 16)

        def start(k, rv, sem):
            return pltpu.async_copy(
                vals_hbm.at[g].at[si_v.at[pl.ds(k * CH, CH)]], rv, sem)

        def process(k, rv):
            def grp(i, carry):
                dvec = di_v[pl.ds(k * CH + i * 16, 16)]
                for j in range(16):
                    d = dvec[j]
                    row = rv[i * 16 + j, :]
                    plsc.addupdate_scatter(acc, [d * 16 + cidx], row)
                return carry

            lax.fori_loop(0, CH // 16, grp, 0)

        for s in range(NSEG):
            base = hf * EPH + s * SEG
            pltpu.sync_copy(src_hbm.at[pl.ds(base, SEG)], si_v)
            pltpu.sync_copy(dst_hbm.at[pl.ds(base, SEG)], di_v)
            start(0, rva, sema)

            def pair(k2, carry):
                k = 2 * k2
                start(k + 1, rvb, semb)
                pltpu.make_async_copy(
                    vals_hbm.at[g].at[si_v.at[pl.ds(0, CH)]], rva, sema).wait()
                process(k, rva)
                start(jnp.minimum(k + 2, CPS - 1), rva, sema)
                pltpu.make_async_copy(
                    vals_hbm.at[g].at[si_v.at[pl.ds(0, CH)]], rvb, semb).wait()
                process(k + 1, rvb)
                return carry

            lax.fori_loop(0, CPS // 2, pair, 0)
            pltpu.make_async_copy(
                vals_hbm.at[g].at[si_v.at[pl.ds(0, CH)]], rva, sema).wait()

        pltpu.sync_copy(acc, out_hbm.at[hf, g])

    return seg_kernel(vals_t, src, dst, zeros)


def _sc_gather(table, idx_flat):
    B = idx_flat.shape[0]
    bpw = B // NW
    nch = bpw // CH

    @functools.partial(
        pl.kernel,
        out_type=jax.ShapeDtypeStruct((B, H), jnp.float32),
        mesh=_sc_mesh(),
        compiler_params=pltpu.CompilerParams(needs_layout_passes=False),
        scratch_types=[
            pltpu.VMEM((CH,), jnp.int32),
            pltpu.VMEM((CH, H), jnp.float32),
            pltpu.SemaphoreType.DMA,
        ],
    )
    def gather_kernel(table_hbm, idx_hbm, out_hbm, idx_v, rows_v, sem):
        wid = _worker_id()

        def chunk(k, carry):
            base = wid * bpw + k * CH
            pltpu.sync_copy(idx_hbm.at[pl.ds(base, CH)], idx_v)
            pltpu.async_copy(table_hbm.at[idx_v], rows_v, sem).wait()
            pltpu.sync_copy(rows_v, out_hbm.at[pl.ds(base, CH)])
            return carry

        lax.fori_loop(0, nch, chunk, 0)

    return gather_kernel(table, idx_flat)


def _tc_prep(degp, x, conv1_W):
    BR = 256

    def body(dp_ref, x_ref, w0_ref, w1_ref, xw_ref, dinv_ref):
        deg = jnp.sum(dp_ref[...], axis=1, keepdims=True) + 1.0
        dinv = 1.0 / jnp.sqrt(deg)
        w = jnp.concatenate([w0_ref[...], w1_ref[...]], axis=0)
        xw = jnp.dot(x_ref[...], w, preferred_element_type=jnp.float32)
        xw_ref[...] = xw * dinv
        dinv_ref[...] = jnp.broadcast_to(dinv, (BR, 128))

    return pl.pallas_call(
        body,
        grid=(N // BR,),
        in_specs=[
            pl.BlockSpec((BR, 128), lambda i: (i, 0)),
            pl.BlockSpec((BR, 2), lambda i: (i, 0)),
            pl.BlockSpec((1, H), lambda i: (0, 0)),
            pl.BlockSpec((1, H), lambda i: (0, 0)),
        ],
        out_specs=[
            pl.BlockSpec((BR, H), lambda i: (i, 0)),
            pl.BlockSpec((BR, 128), lambda i: (i, 0)),
        ],
        out_shape=[
            jax.ShapeDtypeStruct((N, H), jnp.float32),
            jax.ShapeDtypeStruct((N, 128), jnp.float32),
        ],
    )(degp, x, conv1_W[0:1], conv1_W[1:2])


def _tc_layer(sp, xws, dinv, b, W):
    BR = 256

    def body(sp_ref, xws_ref, dinv_ref, b_ref, w_ref, out_ref):
        dinv = dinv_ref[...]
        h = jnp.maximum(dinv * (sp_ref[0] + sp_ref[1] + xws_ref[...]) + b_ref[...], 0.0)
        out_ref[...] = jnp.dot(h, w_ref[...], preferred_element_type=jnp.float32) * dinv

    return pl.pallas_call(
        body,
        grid=(N // BR,),
        in_specs=[
            pl.BlockSpec((NC, BR, H), lambda i: (0, i, 0)),
            pl.BlockSpec((BR, H), lambda i: (i, 0)),
            pl.BlockSpec((BR, 1), lambda i: (i, 0)),
            pl.BlockSpec((1, H), lambda i: (0, 0)),
            pl.BlockSpec((H, H), lambda i: (0, 0)),
        ],
        out_specs=pl.BlockSpec((BR, H), lambda i: (i, 0)),
        out_shape=jax.ShapeDtypeStruct((N, H), jnp.float32),
    )(sp, xws, dinv, b, W)


def _tc_node_mlp(sp, xws, dinv, b2, z2d, w1h, w1z, b1m, w2m, b2m, wsrc, wtgt, wu, wv):
    BR = 256

    def body(sp_ref, xws_ref, dinv_ref, b2_ref, z_ref, w1h_ref, w1z_ref,
             b1m_ref, w2m_ref, b2m_ref, wsrc_ref, wtgt_ref, wu_ref, wv_ref,
             psrc_ref, ptgt_ref, au_ref, bv_ref):
        dinv = dinv_ref[...]
        h2 = jnp.maximum(dinv * (sp_ref[0] + sp_ref[1] + xws_ref[...]) + b2_ref[...], 0.0)
        zw = jnp.dot(z_ref[...], w1z_ref[...], preferred_element_type=jnp.float32)
        t = jnp.maximum(
            jnp.dot(h2, w1h_ref[...], preferred_element_type=jnp.float32)
            + zw + b1m_ref[...], 0.0)
        h = jnp.dot(t, w2m_ref[...], preferred_element_type=jnp.float32) + b2m_ref[...]
        psrc_ref[...] = jnp.dot(h, wsrc_ref[...], preferred_element_type=jnp.float32)
        ptgt_ref[...] = jnp.dot(h, wtgt_ref[...], preferred_element_type=jnp.float32)
        au_ref[...] = jnp.dot(h, wu_ref[...], preferred_element_type=jnp.float32)
        bv_ref[...] = jnp.dot(h, wv_ref[...], preferred_element_type=jnp.float32)

    full = lambda shape: pl.BlockSpec(shape, lambda i: tuple(0 for _ in shape))
    row = lambda m: pl.BlockSpec((BR, m), lambda i: (i, 0))
    return pl.pallas_call(
        body,
        grid=(N // BR,),
        in_specs=[
            pl.BlockSpec((NC, BR, H), lambda i: (0, i, 0)),
            row(H), row(1), full((1, H)), full((1, Z)),
            full((H, H)), full((Z, H)), full((1, H)), full((H, H)), full((1, H)),
            full((H, H)), full((H, H)), full((H, H)), full((H, H)),
        ],
        out_specs=[row(H), row(H), row(H), row(H)],
        out_shape=[jax.ShapeDtypeStruct((N, H), jnp.float32)] * 4,
    )(sp, xws, dinv, b2, z2d, w1h, w1z, b1m, w2m, b2m, wsrc, wtgt, wu, wv)


def _tc_topk(ptgt, psrc, depth_r, depth_c):
    BR = 128

    def body(pt_ref, ps_ref, dr_ref, dc_ref, vals_ref, idx_ref):
        s = lax.dot_general(pt_ref[...], ps_ref[...],
                            (((1,), (1,)), ((), ())),
                            preferred_element_type=jnp.float32)
        valid = dc_ref[...] < dr_ref[...]
        s = jnp.where(valid, s, jnp.float32(NEG))
        cols = lax.broadcasted_iota(jnp.int32, s.shape, 1)
        m1 = jnp.max(s, axis=1, keepdims=True)
        i1 = jnp.min(jnp.where(s == m1, cols, N), axis=1, keepdims=True)
        s2 = jnp.where(cols == i1, jnp.float32(-3.4e38), s)
        m2 = jnp.max(s2, axis=1, keepdims=True)
        i2 = jnp.min(jnp.where(s2 == m2, cols, N), axis=1, keepdims=True)
        vals_ref[...] = jnp.concatenate([m1, m2], axis=1)
        idx_ref[...] = jnp.concatenate([i1, i2], axis=1)

    return pl.pallas_call(
        body,
        grid=(N // BR,),
        in_specs=[
            pl.BlockSpec((BR, H), lambda i: (i, 0)),
            pl.BlockSpec((N, H), lambda i: (0, 0)),
            pl.BlockSpec((BR, 1), lambda i: (i, 0)),
            pl.BlockSpec((1, N), lambda i: (0, 0)),
        ],
        out_specs=[
            pl.BlockSpec((BR, 2), lambda i: (i, 0)),
            pl.BlockSpec((BR, 2), lambda i: (i, 0)),
        ],
        out_shape=[
            jax.ShapeDtypeStruct((N, 2), jnp.float32),
            jax.ShapeDtypeStruct((N, 2), jnp.int32),
        ],
    )(ptgt, psrc, depth_r, depth_c)


def _tc_edge(ag, bv, z2d, wz, b1, w2, b2s, vals, idx, x, depth_r):
    BR = 256

    def body(ag_ref, bv_ref, z_ref, wz_ref, b1_ref, w2_ref, b2_ref,
             vals_ref, idx_ref, x_ref, dr_ref,
             tk_ref, ip_ref, srco_ref, dsto_ref, attr_ref):
        zc = jnp.dot(z_ref[...], wz_ref[...], preferred_element_type=jnp.float32)
        base = b1_ref[...] + zc
        types = x_ref[:, 0:1].astype(jnp.int32)
        kv = jnp.where(types == 2, 2, 1)
        is_t = (types != 0) & (dr_ref[...] >= 1)
        rowid = (pl.program_id(0) * BR
                 + lax.broadcasted_iota(jnp.int32, (BR, 1), 0))
        tks, ips, srcs, dsts, attrs = [], [], [], [], []
        for j in range(2):
            hid = jnp.maximum(ag_ref[j] + bv_ref[...] + base, 0.0)
            logit = jnp.dot(hid, w2_ref[...], preferred_element_type=jnp.float32) + b2_ref[...]
            prob = jax.nn.sigmoid(logit)
            vj = vals_ref[:, j:j + 1]
            mj = is_t & (kv > j) & (vj > jnp.float32(NEG * 0.5))
            mf = mj.astype(jnp.float32)
            tks.append(vj * mf)
            ips.append(prob * mf)
            srcs.append(jnp.where(mj, idx_ref[:, j:j + 1], -1))
            dsts.append(jnp.where(mj, rowid, -1))
            attrs.append(jnp.where(mj, (prob > 0.5).astype(jnp.int32), 0))
        tk_ref[...] = jnp.concatenate(tks, axis=1)
        ip_ref[...] = jnp.concatenate(ips, axis=1)
        srco_ref[...] = jnp.concatenate(srcs, axis=1)
        dsto_ref[...] = jnp.concatenate(dsts, axis=1)
        attr_ref[...] = jnp.concatenate(attrs, axis=1)

    full = lambda shape: pl.BlockSpec(shape, lambda i: tuple(0 for _ in shape))
    row2 = pl.BlockSpec((BR, 2), lambda i: (i, 0))
    return pl.pallas_call(
        body,
        grid=(N // BR,),
        in_specs=[
            pl.BlockSpec((2, BR, H), lambda i: (0, i, 0)),
            pl.BlockSpec((BR, H), lambda i: (i, 0)),
            full((1, Z)), full((Z, H)), full((1, H)), full((H, 1)), full((1, 1)),
            row2, row2, row2,
            pl.BlockSpec((BR, 1), lambda i: (i, 0)),
        ],
        out_specs=[row2, row2, row2, row2, row2],
        out_shape=[
            jax.ShapeDtypeStruct((N, 2), jnp.float32),
            jax.ShapeDtypeStruct((N, 2), jnp.float32),
            jax.ShapeDtypeStruct((N, 2), jnp.int32),
            jax.ShapeDtypeStruct((N, 2), jnp.int32),
            jax.ShapeDtypeStruct((N, 2), jnp.int32),
        ],
    )(ag, bv, z2d, wz, b1, w2, b2s, vals, idx, x, depth_r)


def kernel(x, edge_index, node_depth, z, conv1_W, conv1_b, conv2_W, conv2_b,
           mlp_W1, mlp_b1, mlp_W2, mlp_b2, Wsrc, Wtgt,
           inv_W1, inv_b1, inv_W2, inv_b2):
    src = edge_index[0]
    dst = edge_index[1]
    depth_r = node_depth.reshape(N, 1)
    depth_c = node_depth.reshape(1, N)
    z2d = z.reshape(1, Z)

    degp = jnp.pad(_sc_degree(dst), ((0, 128 - NW), (0, 0))).T
    xw1s, dinv = _tc_prep(degp, x, conv1_W)
    dinv = dinv[:, 0:1]
    xw1t = xw1s.reshape(N, 16, 16).transpose(1, 0, 2)
    sp1 = (_sc_segsum(xw1t, src, dst).reshape(2, 16, N, 16)
           .transpose(0, 2, 1, 3).reshape(2, N, H))
    xw2s = _tc_layer(sp1, xw1s, dinv, conv1_b.reshape(1, H), conv2_W)
    xw2t = xw2s.reshape(N, 16, 16).transpose(1, 0, 2)
    sp2 = (_sc_segsum(xw2t, src, dst).reshape(2, 16, N, 16)
           .transpose(0, 2, 1, 3).reshape(2, N, H))
    psrc, ptgt, a_u, b_v = _tc_node_mlp(
        sp2, xw2s, dinv, conv2_b.reshape(1, H), z2d,
        mlp_W1[:H], mlp_W1[H:], mlp_b1.reshape(1, H), mlp_W2,
        mlp_b2.reshape(1, H), Wsrc, Wtgt, inv_W1[:H], inv_W1[H:2 * H])
    vals, idx = _tc_topk(ptgt, psrc, depth_r, depth_c)
    idx_flat = jnp.concatenate([idx[:, 0], idx[:, 1]], axis=0)
    ag = _sc_gather(a_u, idx_flat).reshape(2, N, H)
    tk, ip, srco, dsto, attr = _tc_edge(
        ag, b_v, z2d, inv_W1[2 * H:], inv_b1.reshape(1, H),
        inv_W2, inv_b2.reshape(1, 1), vals, idx, x, depth_r)
    edge_index_out = jnp.stack([srco.reshape(-1), dsto.reshape(-1)])
    return edge_index_out, attr.reshape(-1), tk, ip


# speedup vs baseline: 3.8901x; 1.0419x over previous
import functools

import jax
import jax.numpy as jnp
from jax import lax
from jax.experimental import pallas as pl
from jax.experimental.pallas import tpu as pltpu
from jax.experimental.pallas import tpu_sc as plsc

N = 4096
E = 65536
H = 256
Z = 128
NEG = -1e9

NC = 2
NS = 16
NW = NC * NS
CH = 128
EPW = E // NW
NCHUNK = EPW // CH
RPT = N // NS

def _sc_mesh():
    return plsc.VectorSubcoreMesh(
        core_axis_name="c", subcore_axis_name="s",
        num_cores=NC, num_subcores=NS)


def _worker_id():
    return lax.axis_index("s") * NC + lax.axis_index("c")


def _sc_degree(dst):
    zeros = jnp.zeros((N,), jnp.float32)

    @functools.partial(
        pl.kernel,
        out_type=jax.ShapeDtypeStruct((NW, N), jnp.float32),
        mesh=_sc_mesh(),
        compiler_params=pltpu.CompilerParams(needs_layout_passes=False),
        scratch_types=[
            pltpu.VMEM((EPW,), jnp.int32),
            pltpu.VMEM((N,), jnp.float32),
        ],
    )
    def deg_kernel(dst_hbm, zeros_hbm, out_hbm, idx_v, acc):
        wid = _worker_id()
        pltpu.sync_copy(zeros_hbm, acc)
        pltpu.sync_copy(dst_hbm.at[pl.ds(wid * EPW, EPW)], idx_v)
        ones = jnp.ones((16,), jnp.float32)

        def grp(i, carry):
            dvec = idx_v[pl.ds(i * 16, 16)]
            plsc.addupdate_scatter(acc, [dvec], ones)
            return carry

        lax.fori_loop(0, EPW // 16, grp, 0)
        pltpu.sync_copy(acc, out_hbm.at[wid])

    return deg_kernel(dst, zeros)


EPH = E // 2
SEG = 8192
NSEG = EPH // SEG
CPS = SEG // CH


def _sc_segsum(vals_t, src, dst):
    zeros = jnp.zeros((N * 16,), jnp.float32)

    @functools.partial(
        pl.kernel,
        out_type=jax.ShapeDtypeStruct((2, 16, N * 16), jnp.float32),
        mesh=_sc_mesh(),
        compiler_params=pltpu.CompilerParams(
            needs_layout_passes=False, use_tc_tiling_on_sc=False),
        scratch_types=[
            pltpu.VMEM((SEG,), jnp.int32),
            pltpu.VMEM((SEG,), jnp.int32),
            pltpu.VMEM((CH, 16), jnp.float32),
            pltpu.VMEM((CH, 16), jnp.float32),
            pltpu.VMEM((N * 16,), jnp.float32),
            pltpu.SemaphoreType.DMA,
            pltpu.SemaphoreType.DMA,
        ],
    )
    def seg_kernel(vals_hbm, src_hbm, dst_hbm, zeros_hbm, out_hbm,
                   si_v, di_v, rva, rvb, acc, sema, semb):
        wid = _worker_id()
        g = wid % 16
        hf = wid // 16
        pltpu.sync_copy(zeros_hbm, acc)
        cidx = lax.iota(jnp.int32, 16)

        def start(k, rv, sem):
            return pltpu.async_copy(
                vals_hbm.at[g].at[si_v.at[pl.ds(k * CH, CH)]], rv, sem)

        def process(k, rv):
            def grp(i, carry):
                dvec = di_v[pl.ds(k * CH + i * 16, 16)]
                for j in range(16):
                    d = dvec[j]
                    row = rv[i * 16 + j, :]
                    plsc.addupdate_scatter(acc, [d * 16 + cidx], row)
                return carry

            lax.fori_loop(0, CH // 16, grp, 0)

        for s in range(NSEG):
            base = hf * EPH + s * SEG
            pltpu.sync_copy(src_hbm.at[pl.ds(base, SEG)], si_v)
            pltpu.sync_copy(dst_hbm.at[pl.ds(base, SEG)], di_v)
            start(0, rva, sema)

            def pair(k2, carry):
                k = 2 * k2
                start(k + 1, rvb, semb)
                pltpu.make_async_copy(
                    vals_hbm.at[g].at[si_v.at[pl.ds(0, CH)]], rva, sema).wait()
                process(k, rva)
                start(jnp.minimum(k + 2, CPS - 1), rva, sema)
                pltpu.make_async_copy(
                    vals_hbm.at[g].at[si_v.at[pl.ds(0, CH)]], rvb, semb).wait()
                process(k + 1, rvb)
                return carry

            lax.fori_loop(0, CPS // 2, pair, 0)
            pltpu.make_async_copy(
                vals_hbm.at[g].at[si_v.at[pl.ds(0, CH)]], rva, sema).wait()

        pltpu.sync_copy(acc, out_hbm.at[hf, g])

    return seg_kernel(vals_t, src, dst, zeros)


def _sc_gather(table, idx_flat):
    B = idx_flat.shape[0]
    bpw = B // NW
    nch = bpw // CH

    @functools.partial(
        pl.kernel,
        out_type=jax.ShapeDtypeStruct((B, H), jnp.float32),
        mesh=_sc_mesh(),
        compiler_params=pltpu.CompilerParams(needs_layout_passes=False),
        scratch_types=[
            pltpu.VMEM((CH,), jnp.int32),
            pltpu.VMEM((CH, H), jnp.float32),
            pltpu.SemaphoreType.DMA,
        ],
    )
    def gather_kernel(table_hbm, idx_hbm, out_hbm, idx_v, rows_v, sem):
        wid = _worker_id()

        def chunk(k, carry):
            base = wid * bpw + k * CH
            pltpu.sync_copy(idx_hbm.at[pl.ds(base, CH)], idx_v)
            pltpu.async_copy(table_hbm.at[idx_v], rows_v, sem).wait()
            pltpu.sync_copy(rows_v, out_hbm.at[pl.ds(base, CH)])
            return carry

        lax.fori_loop(0, nch, chunk, 0)

    return gather_kernel(table, idx_flat)


def _tc_prep(degp, x, conv1_W):
    BR = 256

    def body(dp_ref, x_ref, w0_ref, w1_ref, xw_ref, dinv_ref):
        deg = jnp.sum(dp_ref[...], axis=1, keepdims=True) + 1.0
        dinv = 1.0 / jnp.sqrt(deg)
        w = jnp.concatenate([w0_ref[...], w1_ref[...]], axis=0)
        xw = jnp.dot(x_ref[...], w, preferred_element_type=jnp.float32)
        xw_ref[...] = xw * dinv
        dinv_ref[...] = jnp.broadcast_to(dinv, (BR, 128))

    return pl.pallas_call(
        body,
        grid=(N // BR,),
        in_specs=[
            pl.BlockSpec((BR, 128), lambda i: (i, 0)),
            pl.BlockSpec((BR, 2), lambda i: (i, 0)),
            pl.BlockSpec((1, H), lambda i: (0, 0)),
            pl.BlockSpec((1, H), lambda i: (0, 0)),
        ],
        out_specs=[
            pl.BlockSpec((BR, H), lambda i: (i, 0)),
            pl.BlockSpec((BR, 128), lambda i: (i, 0)),
        ],
        out_shape=[
            jax.ShapeDtypeStruct((N, H), jnp.float32),
            jax.ShapeDtypeStruct((N, 128), jnp.float32),
        ],
    )(degp, x, conv1_W[0:1], conv1_W[1:2])


def _tc_layer(sp, xws, dinv, b, W):
    BR = 256

    def body(sp_ref, xws_ref, dinv_ref, b_ref, w_ref, out_ref):
        dinv = dinv_ref[...]
        h = jnp.maximum(dinv * (sp_ref[0] + sp_ref[1] + xws_ref[...]) + b_ref[...], 0.0)
        out_ref[...] = jnp.dot(h, w_ref[...], preferred_element_type=jnp.float32) * dinv

    return pl.pallas_call(
        body,
        grid=(N // BR,),
        in_specs=[
            pl.BlockSpec((NC, BR, H), lambda i: (0, i, 0)),
            pl.BlockSpec((BR, H), lambda i: (i, 0)),
            pl.BlockSpec((BR, 1), lambda i: (i, 0)),
            pl.BlockSpec((1, H), lambda i: (0, 0)),
            pl.BlockSpec((H, H), lambda i: (0, 0)),
        ],
        out_specs=pl.BlockSpec((BR, H), lambda i: (i, 0)),
        out_shape=jax.ShapeDtypeStruct((N, H), jnp.float32),
    )(sp, xws, dinv, b, W)


def _tc_node_mlp(sp, xws, dinv, b2, z2d, w1h, w1z, b1m, w2m, b2m, wsrc, wtgt, wu, wv):
    BR = 256

    def body(sp_ref, xws_ref, dinv_ref, b2_ref, z_ref, w1h_ref, w1z_ref,
             b1m_ref, w2m_ref, b2m_ref, wsrc_ref, wtgt_ref, wu_ref, wv_ref,
             psrc_ref, ptgt_ref, au_ref, bv_ref):
        dinv = dinv_ref[...]
        h2 = jnp.maximum(dinv * (sp_ref[0] + sp_ref[1] + xws_ref[...]) + b2_ref[...], 0.0)
        zw = jnp.dot(z_ref[...], w1z_ref[...], preferred_element_type=jnp.float32)
        t = jnp.maximum(
            jnp.dot(h2, w1h_ref[...], preferred_element_type=jnp.float32)
            + zw + b1m_ref[...], 0.0)
        h = jnp.dot(t, w2m_ref[...], preferred_element_type=jnp.float32) + b2m_ref[...]
        psrc_ref[...] = jnp.dot(h, wsrc_ref[...], preferred_element_type=jnp.float32)
        ptgt_ref[...] = jnp.dot(h, wtgt_ref[...], preferred_element_type=jnp.float32)
        au_ref[...] = jnp.dot(h, wu_ref[...], preferred_element_type=jnp.float32)
        bv_ref[...] = jnp.dot(h, wv_ref[...], preferred_element_type=jnp.float32)

    full = lambda shape: pl.BlockSpec(shape, lambda i: tuple(0 for _ in shape))
    row = lambda m: pl.BlockSpec((BR, m), lambda i: (i, 0))
    return pl.pallas_call(
        body,
        grid=(N // BR,),
        in_specs=[
            pl.BlockSpec((NC, BR, H), lambda i: (0, i, 0)),
            row(H), row(1), full((1, H)), full((1, Z)),
            full((H, H)), full((Z, H)), full((1, H)), full((H, H)), full((1, H)),
            full((H, H)), full((H, H)), full((H, H)), full((H, H)),
        ],
        out_specs=[row(H), row(H), row(H), row(H)],
        out_shape=[jax.ShapeDtypeStruct((N, H), jnp.float32)] * 4,
    )(sp, xws, dinv, b2, z2d, w1h, w1z, b1m, w2m, b2m, wsrc, wtgt, wu, wv)


def _tc_topk(ptgt, psrc, depth_r, depth_c):
    BR = 128

    def body(pt_ref, ps_ref, dr_ref, dc_ref, vals_ref, idx_ref):
        s = lax.dot_general(pt_ref[...], ps_ref[...],
                            (((1,), (1,)), ((), ())),
                            preferred_element_type=jnp.float32)
        valid = dc_ref[...] < dr_ref[...]
        s = jnp.where(valid, s, jnp.float32(NEG))
        cols = lax.broadcasted_iota(jnp.int32, s.shape, 1)
        m1 = jnp.max(s, axis=1, keepdims=True)
        i1 = jnp.min(jnp.where(s == m1, cols, N), axis=1, keepdims=True)
        s2 = jnp.where(cols == i1, jnp.float32(-3.4e38), s)
        m2 = jnp.max(s2, axis=1, keepdims=True)
        i2 = jnp.min(jnp.where(s2 == m2, cols, N), axis=1, keepdims=True)
        vals_ref[...] = jnp.concatenate([m1, m2], axis=1)
        idx_ref[...] = jnp.concatenate([i1, i2], axis=1)

    return pl.pallas_call(
        body,
        grid=(N // BR,),
        in_specs=[
            pl.BlockSpec((BR, H), lambda i: (i, 0)),
            pl.BlockSpec((N, H), lambda i: (0, 0)),
            pl.BlockSpec((BR, 1), lambda i: (i, 0)),
            pl.BlockSpec((1, N), lambda i: (0, 0)),
        ],
        out_specs=[
            pl.BlockSpec((BR, 2), lambda i: (i, 0)),
            pl.BlockSpec((BR, 2), lambda i: (i, 0)),
        ],
        out_shape=[
            jax.ShapeDtypeStruct((N, 2), jnp.float32),
            jax.ShapeDtypeStruct((N, 2), jnp.int32),
        ],
    )(ptgt, psrc, depth_r, depth_c)


def _tc_edge(ag, bv, z2d, wz, b1, w2, b2s, vals, idx, x, depth_r):
    BR = 256

    def body(ag_ref, bv_ref, z_ref, wz_ref, b1_ref, w2_ref, b2_ref,
             vals_ref, idx_ref, x_ref, dr_ref,
             tk_ref, ip_ref, srco_ref, dsto_ref, attr_ref):
        zc = jnp.dot(z_ref[...], wz_ref[...], preferred_element_type=jnp.float32)
        base = b1_ref[...] + zc
        types = x_ref[:, 0:1].astype(jnp.int32)
        kv = jnp.where(types == 2, 2, 1)
        is_t = (types != 0) & (dr_ref[...] >= 1)
        rowid = (pl.program_id(0) * BR
                 + lax.broadcasted_iota(jnp.int32, (BR, 1), 0))
        tks, ips, srcs, dsts, attrs = [], [], [], [], []
        for j in range(2):
            hid = jnp.maximum(ag_ref[j] + bv_ref[...] + base, 0.0)
            logit = jnp.dot(hid, w2_ref[...], preferred_element_type=jnp.float32) + b2_ref[...]
            prob = jax.nn.sigmoid(logit)
            vj = vals_ref[:, j:j + 1]
            mj = is_t & (kv > j) & (vj > jnp.float32(NEG * 0.5))
            mf = mj.astype(jnp.float32)
            tks.append(vj * mf)
            ips.append(prob * mf)
            srcs.append(jnp.where(mj, idx_ref[:, j:j + 1], -1))
            dsts.append(jnp.where(mj, rowid, -1))
            attrs.append(jnp.where(mj, (prob > 0.5).astype(jnp.int32), 0))
        tk_ref[...] = jnp.concatenate(tks, axis=1)
        ip_ref[...] = jnp.concatenate(ips, axis=1)
        srco_ref[...] = jnp.concatenate(srcs, axis=1)
        dsto_ref[...] = jnp.concatenate(dsts, axis=1)
        attr_ref[...] = jnp.concatenate(attrs, axis=1)

    full = lambda shape: pl.BlockSpec(shape, lambda i: tuple(0 for _ in shape))
    row2 = pl.BlockSpec((BR, 2), lambda i: (i, 0))
    return pl.pallas_call(
        body,
        grid=(N // BR,),
        in_specs=[
            pl.BlockSpec((2, BR, H), lambda i: (0, i, 0)),
            pl.BlockSpec((BR, H), lambda i: (i, 0)),
            full((1, Z)), full((Z, H)), full((1, H)), full((H, 1)), full((1, 1)),
            row2, row2, row2,
            pl.BlockSpec((BR, 1), lambda i: (i, 0)),
        ],
        out_specs=[row2, row2, row2, row2, row2],
        out_shape=[
            jax.ShapeDtypeStruct((N, 2), jnp.float32),
            jax.ShapeDtypeStruct((N, 2), jnp.float32),
            jax.ShapeDtypeStruct((N, 2), jnp.int32),
            jax.ShapeDtypeStruct((N, 2), jnp.int32),
            jax.ShapeDtypeStruct((N, 2), jnp.int32),
        ],
    )(ag, bv, z2d, wz, b1, w2, b2s, vals, idx, x, depth_r)


def kernel(x, edge_index, node_depth, z, conv1_W, conv1_b, conv2_W, conv2_b,
           mlp_W1, mlp_b1, mlp_W2, mlp_b2, Wsrc, Wtgt,
           inv_W1, inv_b1, inv_W2, inv_b2):
    src = edge_index[0]
    dst = edge_index[1]
    depth_r = node_depth.reshape(N, 1)
    depth_c = node_depth.reshape(1, N)
    z2d = z.reshape(1, Z)

    degp = jnp.pad(_sc_degree(dst), ((0, 128 - NW), (0, 0))).T
    xw1s, dinv = _tc_prep(degp, x, conv1_W)
    dinv = dinv[:, 0:1]
    xw1t = xw1s.reshape(N, 16, 16).transpose(1, 0, 2)
    sp1 = (_sc_segsum(xw1t, src, dst).reshape(2, 16, N, 16)
           .transpose(0, 2, 1, 3).reshape(2, N, H))
    xw2s = _tc_layer(sp1, xw1s, dinv, conv1_b.reshape(1, H), conv2_W)
    xw2t = xw2s.reshape(N, 16, 16).transpose(1, 0, 2)
    sp2 = (_sc_segsum(xw2t, src, dst).reshape(2, 16, N, 16)
           .transpose(0, 2, 1, 3).reshape(2, N, H))
    psrc, ptgt, a_u, b_v = _tc_node_mlp(
        sp2, xw2s, dinv, conv2_b.reshape(1, H), z2d,
        mlp_W1[:H], mlp_W1[H:], mlp_b1.reshape(1, H), mlp_W2,
        mlp_b2.reshape(1, H), Wsrc, Wtgt, inv_W1[:H], inv_W1[H:2 * H])
    vals, idx = _tc_topk(ptgt, psrc, depth_r, depth_c)
    idx_flat = jnp.concatenate([idx[:, 0], idx[:, 1]], axis=0)
    ag = _sc_gather(a_u, idx_flat).reshape(2, N, H)
    tk, ip, srco, dsto, attr = _tc_edge(
        ag, b_v, z2d, inv_W1[2 * H:], inv_b1.reshape(1, H),
        inv_W2, inv_b2.reshape(1, 1), vals, idx, x, depth_r)
    edge_index_out = jnp.stack([srco.reshape(-1), dsto.reshape(-1)])
    return edge_index_out, attr.reshape(-1), tk, ip


# speedup vs baseline: 3.8943x; 1.0011x over previous
import functools

import jax
import jax.numpy as jnp
from jax import lax
from jax.experimental import pallas as pl
from jax.experimental.pallas import tpu as pltpu
from jax.experimental.pallas import tpu_sc as plsc

N = 4096
E = 65536
H = 256
Z = 128
NEG = -1e9

NC = 2
NS = 16
NW = NC * NS
CH = 128
EPW = E // NW
NCHUNK = EPW // CH
RPT = N // NS

def _sc_mesh():
    return plsc.VectorSubcoreMesh(
        core_axis_name="c", subcore_axis_name="s",
        num_cores=NC, num_subcores=NS)


def _worker_id():
    return lax.axis_index("s") * NC + lax.axis_index("c")


def _sc_degree(dst):
    zeros = jnp.zeros((N,), jnp.float32)

    @functools.partial(
        pl.kernel,
        out_type=jax.ShapeDtypeStruct((NW, N), jnp.float32),
        mesh=_sc_mesh(),
        compiler_params=pltpu.CompilerParams(needs_layout_passes=False),
        scratch_types=[
            pltpu.VMEM((EPW,), jnp.int32),
            pltpu.VMEM((N,), jnp.float32),
        ],
    )
    def deg_kernel(dst_hbm, zeros_hbm, out_hbm, idx_v, acc):
        wid = _worker_id()
        pltpu.sync_copy(zeros_hbm, acc)
        pltpu.sync_copy(dst_hbm.at[pl.ds(wid * EPW, EPW)], idx_v)
        ones = jnp.ones((16,), jnp.float32)

        def grp(i, carry):
            dvec = idx_v[pl.ds(i * 16, 16)]
            plsc.addupdate_scatter(acc, [dvec], ones)
            return carry

        lax.fori_loop(0, EPW // 16, grp, 0)
        pltpu.sync_copy(acc, out_hbm.at[wid])

    return deg_kernel(dst, zeros)


EPH = E // 2
SEG = 8192
NSEG = EPH // SEG
CPS = SEG // CH


def _sc_segsum(vals_t, src, dst):
    zeros = jnp.zeros((N * 16,), jnp.float32)

    @functools.partial(
        pl.kernel,
        out_type=jax.ShapeDtypeStruct((2, 16, N * 16), jnp.float32),
        mesh=_sc_mesh(),
        compiler_params=pltpu.CompilerParams(
            needs_layout_passes=False, use_tc_tiling_on_sc=False),
        scratch_types=[
            pltpu.VMEM((SEG,), jnp.int32),
            pltpu.VMEM((SEG,), jnp.int32),
            pltpu.VMEM((CH, 16), jnp.float32),
            pltpu.VMEM((CH, 16), jnp.float32),
            pltpu.VMEM((N * 16,), jnp.float32),
            pltpu.SemaphoreType.DMA,
            pltpu.SemaphoreType.DMA,
        ],
    )
    def seg_kernel(vals_hbm, src_hbm, dst_hbm, zeros_hbm, out_hbm,
                   si_v, di_v, rva, rvb, acc, sema, semb):
        wid = _worker_id()
        g = wid % 16
        hf = wid // 16
        pltpu.sync_copy(zeros_hbm, acc)
        cidx = lax.iota(jnp.int32, 16)

        def start(k, rv, sem):
            return pltpu.async_copy(
                vals_hbm.at[g].at[si_v.at[pl.ds(k * CH, CH)]], rv, sem)

        def process(k, rv):
            def grp(i, carry):
                advec = di_v[pl.ds(k * CH + i * 16, 16)] * 16
                for j in range(16):
                    row = rv[i * 16 + j, :]
                    plsc.addupdate_scatter(acc, [advec[j] + cidx], row)
                return carry

            lax.fori_loop(0, CH // 16, grp, 0)

        for s in range(NSEG):
            base = hf * EPH + s * SEG
            pltpu.sync_copy(src_hbm.at[pl.ds(base, SEG)], si_v)
            pltpu.sync_copy(dst_hbm.at[pl.ds(base, SEG)], di_v)
            start(0, rva, sema)

            def pair(k2, carry):
                k = 2 * k2
                start(k + 1, rvb, semb)
                pltpu.make_async_copy(
                    vals_hbm.at[g].at[si_v.at[pl.ds(0, CH)]], rva, sema).wait()
                process(k, rva)
                start(jnp.minimum(k + 2, CPS - 1), rva, sema)
                pltpu.make_async_copy(
                    vals_hbm.at[g].at[si_v.at[pl.ds(0, CH)]], rvb, semb).wait()
                process(k + 1, rvb)
                return carry

            lax.fori_loop(0, CPS // 2, pair, 0)
            pltpu.make_async_copy(
                vals_hbm.at[g].at[si_v.at[pl.ds(0, CH)]], rva, sema).wait()

        pltpu.sync_copy(acc, out_hbm.at[hf, g])

    return seg_kernel(vals_t, src, dst, zeros)


def _sc_gather(table, idx_flat):
    B = idx_flat.shape[0]
    bpw = B // NW
    nch = bpw // CH

    @functools.partial(
        pl.kernel,
        out_type=jax.ShapeDtypeStruct((B, H), jnp.float32),
        mesh=_sc_mesh(),
        compiler_params=pltpu.CompilerParams(needs_layout_passes=False),
        scratch_types=[
            pltpu.VMEM((CH,), jnp.int32),
            pltpu.VMEM((CH, H), jnp.float32),
            pltpu.SemaphoreType.DMA,
        ],
    )
    def gather_kernel(table_hbm, idx_hbm, out_hbm, idx_v, rows_v, sem):
        wid = _worker_id()

        def chunk(k, carry):
            base = wid * bpw + k * CH
            pltpu.sync_copy(idx_hbm.at[pl.ds(base, CH)], idx_v)
            pltpu.async_copy(table_hbm.at[idx_v], rows_v, sem).wait()
            pltpu.sync_copy(rows_v, out_hbm.at[pl.ds(base, CH)])
            return carry

        lax.fori_loop(0, nch, chunk, 0)

    return gather_kernel(table, idx_flat)


def _tc_prep(degp, x, conv1_W):
    BR = 256

    def body(dp_ref, x_ref, w0_ref, w1_ref, xw_ref, dinv_ref):
        deg = jnp.sum(dp_ref[...], axis=1, keepdims=True) + 1.0
        dinv = 1.0 / jnp.sqrt(deg)
        w = jnp.concatenate([w0_ref[...], w1_ref[...]], axis=0)
        xw = jnp.dot(x_ref[...], w, preferred_element_type=jnp.float32)
        xw_ref[...] = xw * dinv
        dinv_ref[...] = jnp.broadcast_to(dinv, (BR, 128))

    return pl.pallas_call(
        body,
        grid=(N // BR,),
        in_specs=[
            pl.BlockSpec((BR, 128), lambda i: (i, 0)),
            pl.BlockSpec((BR, 2), lambda i: (i, 0)),
            pl.BlockSpec((1, H), lambda i: (0, 0)),
            pl.BlockSpec((1, H), lambda i: (0, 0)),
        ],
        out_specs=[
            pl.BlockSpec((BR, H), lambda i: (i, 0)),
            pl.BlockSpec((BR, 128), lambda i: (i, 0)),
        ],
        out_shape=[
            jax.ShapeDtypeStruct((N, H), jnp.float32),
            jax.ShapeDtypeStruct((N, 128), jnp.float32),
        ],
    )(degp, x, conv1_W[0:1], conv1_W[1:2])


def _tc_layer(sp, xws, dinv, b, W):
    BR = 256

    def body(sp_ref, xws_ref, dinv_ref, b_ref, w_ref, out_ref):
        dinv = dinv_ref[...]
        h = jnp.maximum(dinv * (sp_ref[0] + sp_ref[1] + xws_ref[...]) + b_ref[...], 0.0)
        out_ref[...] = jnp.dot(h, w_ref[...], preferred_element_type=jnp.float32) * dinv

    return pl.pallas_call(
        body,
        grid=(N // BR,),
        in_specs=[
            pl.BlockSpec((NC, BR, H), lambda i: (0, i, 0)),
            pl.BlockSpec((BR, H), lambda i: (i, 0)),
            pl.BlockSpec((BR, 1), lambda i: (i, 0)),
            pl.BlockSpec((1, H), lambda i: (0, 0)),
            pl.BlockSpec((H, H), lambda i: (0, 0)),
        ],
        out_specs=pl.BlockSpec((BR, H), lambda i: (i, 0)),
        out_shape=jax.ShapeDtypeStruct((N, H), jnp.float32),
    )(sp, xws, dinv, b, W)


def _tc_node_mlp(sp, xws, dinv, b2, z2d, w1h, w1z, b1m, w2m, b2m, wsrc, wtgt, wu, wv):
    BR = 256

    def body(sp_ref, xws_ref, dinv_ref, b2_ref, z_ref, w1h_ref, w1z_ref,
             b1m_ref, w2m_ref, b2m_ref, wsrc_ref, wtgt_ref, wu_ref, wv_ref,
             psrc_ref, ptgt_ref, au_ref, bv_ref):
        dinv = dinv_ref[...]
        h2 = jnp.maximum(dinv * (sp_ref[0] + sp_ref[1] + xws_ref[...]) + b2_ref[...], 0.0)
        zw = jnp.dot(z_ref[...], w1z_ref[...], preferred_element_type=jnp.float32)
        t = jnp.maximum(
            jnp.dot(h2, w1h_ref[...], preferred_element_type=jnp.float32)
            + zw + b1m_ref[...], 0.0)
        h = jnp.dot(t, w2m_ref[...], preferred_element_type=jnp.float32) + b2m_ref[...]
        psrc_ref[...] = jnp.dot(h, wsrc_ref[...], preferred_element_type=jnp.float32)
        ptgt_ref[...] = jnp.dot(h, wtgt_ref[...], preferred_element_type=jnp.float32)
        au_ref[...] = jnp.dot(h, wu_ref[...], preferred_element_type=jnp.float32)
        bv_ref[...] = jnp.dot(h, wv_ref[...], preferred_element_type=jnp.float32)

    full = lambda shape: pl.BlockSpec(shape, lambda i: tuple(0 for _ in shape))
    row = lambda m: pl.BlockSpec((BR, m), lambda i: (i, 0))
    return pl.pallas_call(
        body,
        grid=(N // BR,),
        in_specs=[
            pl.BlockSpec((NC, BR, H), lambda i: (0, i, 0)),
            row(H), row(1), full((1, H)), full((1, Z)),
            full((H, H)), full((Z, H)), full((1, H)), full((H, H)), full((1, H)),
            full((H, H)), full((H, H)), full((H, H)), full((H, H)),
        ],
        out_specs=[row(H), row(H), row(H), row(H)],
        out_shape=[jax.ShapeDtypeStruct((N, H), jnp.float32)] * 4,
    )(sp, xws, dinv, b2, z2d, w1h, w1z, b1m, w2m, b2m, wsrc, wtgt, wu, wv)


def _tc_topk(ptgt, psrc, depth_r, depth_c):
    BR = 128

    def body(pt_ref, ps_ref, dr_ref, dc_ref, vals_ref, idx_ref):
        s = lax.dot_general(pt_ref[...], ps_ref[...],
                            (((1,), (1,)), ((), ())),
                            preferred_element_type=jnp.float32)
        valid = dc_ref[...] < dr_ref[...]
        s = jnp.where(valid, s, jnp.float32(NEG))
        cols = lax.broadcasted_iota(jnp.int32, s.shape, 1)
        m1 = jnp.max(s, axis=1, keepdims=True)
        i1 = jnp.min(jnp.where(s == m1, cols, N), axis=1, keepdims=True)
        s2 = jnp.where(cols == i1, jnp.float32(-3.4e38), s)
        m2 = jnp.max(s2, axis=1, keepdims=True)
        i2 = jnp.min(jnp.where(s2 == m2, cols, N), axis=1, keepdims=True)
        vals_ref[...] = jnp.concatenate([m1, m2], axis=1)
        idx_ref[...] = jnp.concatenate([i1, i2], axis=1)

    return pl.pallas_call(
        body,
        grid=(N // BR,),
        in_specs=[
            pl.BlockSpec((BR, H), lambda i: (i, 0)),
            pl.BlockSpec((N, H), lambda i: (0, 0)),
            pl.BlockSpec((BR, 1), lambda i: (i, 0)),
            pl.BlockSpec((1, N), lambda i: (0, 0)),
        ],
        out_specs=[
            pl.BlockSpec((BR, 2), lambda i: (i, 0)),
            pl.BlockSpec((BR, 2), lambda i: (i, 0)),
        ],
        out_shape=[
            jax.ShapeDtypeStruct((N, 2), jnp.float32),
            jax.ShapeDtypeStruct((N, 2), jnp.int32),
        ],
    )(ptgt, psrc, depth_r, depth_c)


def _tc_edge(ag, bv, z2d, wz, b1, w2, b2s, vals, idx, x, depth_r):
    BR = 256

    def body(ag_ref, bv_ref, z_ref, wz_ref, b1_ref, w2_ref, b2_ref,
             vals_ref, idx_ref, x_ref, dr_ref,
             tk_ref, ip_ref, srco_ref, dsto_ref, attr_ref):
        zc = jnp.dot(z_ref[...], wz_ref[...], preferred_element_type=jnp.float32)
        base = b1_ref[...] + zc
        types = x_ref[:, 0:1].astype(jnp.int32)
        kv = jnp.where(types == 2, 2, 1)
        is_t = (types != 0) & (dr_ref[...] >= 1)
        rowid = (pl.program_id(0) * BR
                 + lax.broadcasted_iota(jnp.int32, (BR, 1), 0))
        tks, ips, srcs, dsts, attrs = [], [], [], [], []
        for j in range(2):
            hid = jnp.maximum(ag_ref[j] + bv_ref[...] + base, 0.0)
            logit = jnp.dot(hid, w2_ref[...], preferred_element_type=jnp.float32) + b2_ref[...]
            prob = jax.nn.sigmoid(logit)
            vj = vals_ref[:, j:j + 1]
            mj = is_t & (kv > j) & (vj > jnp.float32(NEG * 0.5))
            mf = mj.astype(jnp.float32)
            tks.append(vj * mf)
            ips.append(prob * mf)
            srcs.append(jnp.where(mj, idx_ref[:, j:j + 1], -1))
            dsts.append(jnp.where(mj, rowid, -1))
            attrs.append(jnp.where(mj, (prob > 0.5).astype(jnp.int32), 0))
        tk_ref[...] = jnp.concatenate(tks, axis=1)
        ip_ref[...] = jnp.concatenate(ips, axis=1)
        srco_ref[...] = jnp.concatenate(srcs, axis=1)
        dsto_ref[...] = jnp.concatenate(dsts, axis=1)
        attr_ref[...] = jnp.concatenate(attrs, axis=1)

    full = lambda shape: pl.BlockSpec(shape, lambda i: tuple(0 for _ in shape))
    row2 = pl.BlockSpec((BR, 2), lambda i: (i, 0))
    return pl.pallas_call(
        body,
        grid=(N // BR,),
        in_specs=[
            pl.BlockSpec((2, BR, H), lambda i: (0, i, 0)),
            pl.BlockSpec((BR, H), lambda i: (i, 0)),
            full((1, Z)), full((Z, H)), full((1, H)), full((H, 1)), full((1, 1)),
            row2, row2, row2,
            pl.BlockSpec((BR, 1), lambda i: (i, 0)),
        ],
        out_specs=[row2, row2, row2, row2, row2],
        out_shape=[
            jax.ShapeDtypeStruct((N, 2), jnp.float32),
            jax.ShapeDtypeStruct((N, 2), jnp.float32),
            jax.ShapeDtypeStruct((N, 2), jnp.int32),
            jax.ShapeDtypeStruct((N, 2), jnp.int32),
            jax.ShapeDtypeStruct((N, 2), jnp.int32),
        ],
    )(ag, bv, z2d, wz, b1, w2, b2s, vals, idx, x, depth_r)


def kernel(x, edge_index, node_depth, z, conv1_W, conv1_b, conv2_W, conv2_b,
           mlp_W1, mlp_b1, mlp_W2, mlp_b2, Wsrc, Wtgt,
           inv_W1, inv_b1, inv_W2, inv_b2):
    src = edge_index[0]
    dst = edge_index[1]
    depth_r = node_depth.reshape(N, 1)
    depth_c = node_depth.reshape(1, N)
    z2d = z.reshape(1, Z)

    degp = jnp.pad(_sc_degree(dst), ((0, 128 - NW), (0, 0))).T
    xw1s, dinv = _tc_prep(degp, x, conv1_W)
    dinv = dinv[:, 0:1]
    xw1t = xw1s.reshape(N, 16, 16).transpose(1, 0, 2)
    sp1 = (_sc_segsum(xw1t, src, dst).reshape(2, 16, N, 16)
           .transpose(0, 2, 1, 3).reshape(2, N, H))
    xw2s = _tc_layer(sp1, xw1s, dinv, conv1_b.reshape(1, H), conv2_W)
    xw2t = xw2s.reshape(N, 16, 16).transpose(1, 0, 2)
    sp2 = (_sc_segsum(xw2t, src, dst).reshape(2, 16, N, 16)
           .transpose(0, 2, 1, 3).reshape(2, N, H))
    psrc, ptgt, a_u, b_v = _tc_node_mlp(
        sp2, xw2s, dinv, conv2_b.reshape(1, H), z2d,
        mlp_W1[:H], mlp_W1[H:], mlp_b1.reshape(1, H), mlp_W2,
        mlp_b2.reshape(1, H), Wsrc, Wtgt, inv_W1[:H], inv_W1[H:2 * H])
    vals, idx = _tc_topk(ptgt, psrc, depth_r, depth_c)
    idx_flat = jnp.concatenate([idx[:, 0], idx[:, 1]], axis=0)
    ag = _sc_gather(a_u, idx_flat).reshape(2, N, H)
    tk, ip, srco, dsto, attr = _tc_edge(
        ag, b_v, z2d, inv_W1[2 * H:], inv_b1.reshape(1, H),
        inv_W2, inv_b2.reshape(1, 1), vals, idx, x, depth_r)
    edge_index_out = jnp.stack([srco.reshape(-1), dsto.reshape(-1)])
    return edge_index_out, attr.reshape(-1), tk, ip
